# baseline (speedup 1.0000x reference)
import functools

import jax
import jax.numpy as jnp
from jax import lax
from jax.experimental import pallas as pl
from jax.experimental.pallas import tpu as pltpu
from jax.experimental.pallas import tpu_sc as plsc

NC, NS = 2, 16
NW = NC * NS
LANES = 16
PB = 128


@functools.cache
def _tpe_kernel(B, L, D):
    NPB = L // PB
    NBH = NW // NPB
    BH = B // NBH
    assert NPB * PB == L and NBH * BH == B and D % LANES == 0
    mesh = plsc.VectorSubcoreMesh(core_axis_name="c", subcore_axis_name="s")

    @functools.partial(
        pl.kernel,
        out_type=jax.ShapeDtypeStruct((B * L, D), jnp.float32),
        mesh=mesh,
        scratch_types=[
            pltpu.VMEM((BH, PB), jnp.int32),
            pltpu.VMEM((PB, D), jnp.float32),
            pltpu.VMEM((PB, D), jnp.float32),
            pltpu.SemaphoreType.DMA,
        ],
        compiler_params=pltpu.CompilerParams(use_tc_tiling_on_sc=False),
    )
    def k(x_hbm, tok_hbm, pos_hbm, out_hbm, idx_v, pos_v, rows_v, sem):
        wid = lax.axis_index("s") * NC + lax.axis_index("c")
        pb = lax.rem(wid, NPB)
        bh = wid // NPB
        pbase = pb * PB
        b0 = bh * BH
        pltpu.sync_copy(pos_hbm.at[pl.ds(pbase, PB)], pos_v)
        pltpu.sync_copy(x_hbm.at[pl.ds(b0, BH), pl.ds(pbase, PB)], idx_v)

        def per_batch(b, carry):
            pltpu.async_copy(tok_hbm.at[idx_v.at[b]], rows_v, sem).wait()

            def add_row(r, c):
                for j in range(D // LANES):
                    sl = pl.ds(j * LANES, LANES)
                    plsc.addupdate(rows_v.at[r, sl], pos_v[r, sl])
                return c

            lax.fori_loop(0, PB, add_row, 0)
            pltpu.sync_copy(rows_v, out_hbm.at[pl.ds((b0 + b) * L + pbase, PB)])
            return carry

        lax.fori_loop(0, BH, per_batch, 0)

    return k


def kernel(x, token_table, pos_table):
    B, L = x.shape
    _, D = token_table.shape
    flat = _tpe_kernel(B, L, D)(x.astype(jnp.int32), token_table, pos_table)
    return flat.reshape(B, L, D)
